# baseline (speedup 1.0000x reference)
import functools

import jax
import jax.numpy as jnp
from jax import lax
from jax.experimental import pallas as pl
from jax.experimental.pallas import tpu as pltpu
from jax.experimental.pallas import tpu_sc as plsc

B = 16384
CTX = 20
D = 64
VOCAB = 1000000
NC = 2
NS = 16
NW = NC * NS
BPW = B // NW
CB = 32
NCHUNK = BPW // CB
PAIRS = CB * CTX
NGRP = PAIRS // 16
IDXW = 128
NIDX = PAIRS // IDXW

TCOLS = 2048


def _tc_relayout(table_t):
    grid = (VOCAB + TCOLS - 1) // TCOLS

    def body(in_ref, out_ref):
        x = in_ref[...]
        eye = jnp.eye(D, dtype=jnp.float32)
        xt = lax.dot_general(x, eye, (((0,), (0,)), ((), ())),
                             preferred_element_type=jnp.float32)
        out_ref[:, 0:D] = xt

    return pl.pallas_call(
        body,
        grid=(grid,),
        in_specs=[pl.BlockSpec((D, TCOLS), lambda i: (0, i))],
        out_specs=pl.BlockSpec((TCOLS, 128), lambda i: (i, 0)),
        out_shape=jax.ShapeDtypeStruct((VOCAB, 128), jnp.float32),
    )(table_t)


def _sc_skipgram(target_flat, ctx_flat, ttab, ctab):
    mesh = plsc.VectorSubcoreMesh(core_axis_name="c", subcore_axis_name="s")

    @functools.partial(
        pl.kernel,
        mesh=mesh,
        compiler_params=pltpu.CompilerParams(
            needs_layout_passes=False, use_tc_tiling_on_sc=True),
        out_type=jax.ShapeDtypeStruct((B * CTX,), jnp.float32),
        scratch_types=[
            pltpu.VMEM((CB,), jnp.int32),
            pltpu.VMEM((PAIRS,), jnp.int32),
            pltpu.VMEM((CB, 128), jnp.float32),
            pltpu.VMEM((PAIRS, 128), jnp.float32),
            pltpu.VMEM((PAIRS,), jnp.float32),
            pltpu.SemaphoreType.DMA,
        ],
    )
    def k(tgt_hbm, ctx_hbm, ttab_hbm, ctab_hbm, out_hbm,
          tidx, cidx, te, ce, ob, sem):
        wid = lax.axis_index("s") * NC + lax.axis_index("c")
        lanes = lax.iota(jnp.int32, 16)

        def chunk_body(c, carry):
            base_b = wid * BPW + c * CB
            pltpu.sync_copy(tgt_hbm.at[pl.ds(base_b, CB)], tidx)
            coff = wid * BPW * CTX + c * PAIRS
            pltpu.sync_copy(ctx_hbm.at[pl.ds(coff, PAIRS)], cidx)

            cp_t = pltpu.async_copy(ttab_hbm.at[tidx], te, sem)
            cps = [
                pltpu.async_copy(ctab_hbm.at[cidx.at[pl.ds(kk * IDXW, IDXW)]],
                                 ce.at[pl.ds(kk * IDXW, IDXW)], sem)
                for kk in range(NIDX)
            ]
            cp_t.wait()
            for cp in cps:
                cp.wait()

            def sg_body(sg, carry2):
                b0 = sg * 4
                accs = [jnp.zeros((16,), jnp.float32) for _ in range(5)]
                for boff in range(4):
                    b = b0 + boff
                    t0 = te[b, pl.ds(0, 16)]
                    t1 = te[b, pl.ds(16, 16)]
                    t2 = te[b, pl.ds(32, 16)]
                    t3 = te[b, pl.ds(48, 16)]
                    for j in range(CTX):
                        p = boff * CTX + j
                        row = b * CTX + j
                        s = (t0 * ce[row, pl.ds(0, 16)]
                             + t1 * ce[row, pl.ds(16, 16)]
                             + t2 * ce[row, pl.ds(32, 16)]
                             + t3 * ce[row, pl.ds(48, 16)])
                        v, l = divmod(p, 16)
                        accs[v] = jnp.where(lanes == l, jnp.sum(s), accs[v])
                for v in range(5):
                    ob[pl.ds(sg * 80 + v * 16, 16)] = accs[v]
                return carry2

            lax.fori_loop(0, CB // 4, sg_body, 0, unroll=False)

            out0 = wid * BPW * CTX + c * PAIRS
            pltpu.sync_copy(ob, out_hbm.at[pl.ds(out0, PAIRS)])
            return carry

        lax.fori_loop(0, NCHUNK, chunk_body, 0, unroll=False)

    return k(target_flat, ctx_flat, ttab, ctab)


def kernel(target, context, target_table, context_table):
    target_flat = target.reshape(B)
    ctx_flat = context.reshape(B * CTX)
    ttab = _tc_relayout(target_table.T)
    ctab = _tc_relayout(context_table.T)
    out_flat = _sc_skipgram(target_flat, ctx_flat, ttab, ctab)
    return out_flat.reshape(B, CTX)


# speedup vs baseline: 1.3995x; 1.3995x over previous
import functools

import jax
import jax.numpy as jnp
from jax import lax
from jax.experimental import pallas as pl
from jax.experimental.pallas import tpu as pltpu
from jax.experimental.pallas import tpu_sc as plsc

B = 16384
CTX = 20
D = 64
VOCAB = 1000000
NC = 2
NS = 16
NW = NC * NS
BPW = B // NW
CB = 32
NCHUNK = BPW // CB
PAIRS = CB * CTX
NGRP = PAIRS // 16
IDXW = 128
NIDX = PAIRS // IDXW

TCOLS = 2048


def _tc_relayout(ttab_t, ctab_t):
    grid = (VOCAB + TCOLS - 1) // TCOLS

    def body(t_ref, c_ref, out_ref):
        eye = jnp.eye(D, dtype=jnp.float32)
        tt = lax.dot_general(t_ref[...], eye, (((0,), (0,)), ((), ())),
                             preferred_element_type=jnp.float32)
        ct = lax.dot_general(c_ref[...], eye, (((0,), (0,)), ((), ())),
                             preferred_element_type=jnp.float32)
        out_ref[...] = jnp.concatenate([tt, ct], axis=1)

    return pl.pallas_call(
        body,
        grid=(grid,),
        in_specs=[pl.BlockSpec((D, TCOLS), lambda i: (0, i)),
                  pl.BlockSpec((D, TCOLS), lambda i: (0, i))],
        out_specs=pl.BlockSpec((TCOLS, 128), lambda i: (i, 0)),
        out_shape=jax.ShapeDtypeStruct((VOCAB, 128), jnp.float32),
    )(ttab_t, ctab_t)


def _sc_skipgram(target_flat, ctx_flat, mtab):
    mesh = plsc.VectorSubcoreMesh(core_axis_name="c", subcore_axis_name="s")

    @functools.partial(
        pl.kernel,
        mesh=mesh,
        compiler_params=pltpu.CompilerParams(
            needs_layout_passes=False, use_tc_tiling_on_sc=True),
        out_type=jax.ShapeDtypeStruct((B * CTX,), jnp.float32),
        scratch_types=[
            pltpu.VMEM((CB,), jnp.int32),
            pltpu.VMEM((PAIRS,), jnp.int32),
            pltpu.VMEM((CB, 128), jnp.float32),
            pltpu.VMEM((PAIRS, 128), jnp.float32),
            pltpu.VMEM((PAIRS,), jnp.float32),
            pltpu.SemaphoreType.DMA,
        ],
    )
    def k(tgt_hbm, ctx_hbm, mtab_hbm, out_hbm,
          tidx, cidx, te, ce, ob, sem):
        wid = lax.axis_index("s") * NC + lax.axis_index("c")
        lanes = lax.iota(jnp.int32, 16)

        def chunk_body(c, carry):
            base_b = wid * BPW + c * CB
            pltpu.sync_copy(tgt_hbm.at[pl.ds(base_b, CB)], tidx)
            coff = wid * BPW * CTX + c * PAIRS
            pltpu.sync_copy(ctx_hbm.at[pl.ds(coff, PAIRS)], cidx)

            cp_t = pltpu.async_copy(mtab_hbm.at[tidx], te, sem)
            cps = [
                pltpu.async_copy(mtab_hbm.at[cidx.at[pl.ds(kk * IDXW, IDXW)]],
                                 ce.at[pl.ds(kk * IDXW, IDXW)], sem)
                for kk in range(NIDX)
            ]
            cp_t.wait()
            for cp in cps:
                cp.wait()

            def sg_body(sg, carry2):
                b0 = sg * 4
                accs = [jnp.zeros((16,), jnp.float32) for _ in range(5)]
                for boff in range(4):
                    b = b0 + boff
                    t0 = te[b, pl.ds(0, 16)]
                    t1 = te[b, pl.ds(16, 16)]
                    t2 = te[b, pl.ds(32, 16)]
                    t3 = te[b, pl.ds(48, 16)]
                    for j in range(CTX):
                        p = boff * CTX + j
                        row = b * CTX + j
                        s = (t0 * ce[row, pl.ds(64, 16)]
                             + t1 * ce[row, pl.ds(80, 16)]
                             + t2 * ce[row, pl.ds(96, 16)]
                             + t3 * ce[row, pl.ds(112, 16)])
                        v, l = divmod(p, 16)
                        accs[v] = jnp.where(lanes == l, jnp.sum(s), accs[v])
                for v in range(5):
                    ob[pl.ds(sg * 80 + v * 16, 16)] = accs[v]
                return carry2

            lax.fori_loop(0, CB // 4, sg_body, 0, unroll=False)

            out0 = wid * BPW * CTX + c * PAIRS
            pltpu.sync_copy(ob, out_hbm.at[pl.ds(out0, PAIRS)])
            return carry

        lax.fori_loop(0, NCHUNK, chunk_body, 0, unroll=False)

    return k(target_flat, ctx_flat, mtab)


def kernel(target, context, target_table, context_table):
    target_flat = target.reshape(B)
    ctx_flat = context.reshape(B * CTX)
    mtab = _tc_relayout(target_table.T, context_table.T)
    out_flat = _sc_skipgram(target_flat, ctx_flat, mtab)
    return out_flat.reshape(B, CTX)


# speedup vs baseline: 2.0765x; 1.4837x over previous
import functools

import jax
import jax.numpy as jnp
from jax import lax
from jax.experimental import pallas as pl
from jax.experimental.pallas import tpu as pltpu
from jax.experimental.pallas import tpu_sc as plsc

B = 16384
CTX = 20
D = 64
VOCAB = 1000000
NC = 2
NS = 16
NW = NC * NS
BPW = B // NW
CB = 32
NCHUNK = BPW // CB
PAIRS = CB * CTX
NGRP = PAIRS // 16
IDXW = 128
NIDX = PAIRS // IDXW

TCOLS = 8192


def _tc_relayout(ttab_t, ctab_t):
    grid = (VOCAB + TCOLS - 1) // TCOLS

    def body(t_ref, c_ref, out_ref):
        eye = jnp.eye(D, dtype=jnp.bfloat16)
        tt = lax.dot_general(t_ref[...].astype(jnp.bfloat16), eye,
                             (((0,), (0,)), ((), ())),
                             preferred_element_type=jnp.float32)
        ct = lax.dot_general(c_ref[...].astype(jnp.bfloat16), eye,
                             (((0,), (0,)), ((), ())),
                             preferred_element_type=jnp.float32)
        out_ref[...] = jnp.concatenate([tt, ct], axis=1)

    return pl.pallas_call(
        body,
        grid=(grid,),
        in_specs=[pl.BlockSpec((D, TCOLS), lambda i: (0, i)),
                  pl.BlockSpec((D, TCOLS), lambda i: (0, i))],
        out_specs=pl.BlockSpec((TCOLS, 128), lambda i: (i, 0)),
        out_shape=jax.ShapeDtypeStruct((VOCAB, 128), jnp.float32),
    )(ttab_t, ctab_t)


def _sc_skipgram(target_flat, ctx_flat, mtab):
    mesh = plsc.VectorSubcoreMesh(core_axis_name="c", subcore_axis_name="s")

    @functools.partial(
        pl.kernel,
        mesh=mesh,
        compiler_params=pltpu.CompilerParams(
            needs_layout_passes=False, use_tc_tiling_on_sc=True),
        out_type=jax.ShapeDtypeStruct((B * CTX,), jnp.float32),
        scratch_types=[
            pltpu.VMEM((CB,), jnp.int32),
            pltpu.VMEM((PAIRS,), jnp.int32),
            pltpu.VMEM((CB, 128), jnp.float32),
            pltpu.VMEM((PAIRS, 128), jnp.float32),
            pltpu.VMEM((PAIRS,), jnp.float32),
            pltpu.SemaphoreType.DMA,
        ],
    )
    def k(tgt_hbm, ctx_hbm, mtab_hbm, out_hbm,
          tidx, cidx, te, ce, ob, sem):
        wid = lax.axis_index("s") * NC + lax.axis_index("c")
        lanes = lax.iota(jnp.int32, 16)

        def chunk_body(c, carry):
            base_b = wid * BPW + c * CB
            pltpu.sync_copy(tgt_hbm.at[pl.ds(base_b, CB)], tidx)
            coff = wid * BPW * CTX + c * PAIRS
            pltpu.sync_copy(ctx_hbm.at[pl.ds(coff, PAIRS)], cidx)

            cp_t = pltpu.async_copy(mtab_hbm.at[tidx], te, sem)
            cps = [
                pltpu.async_copy(mtab_hbm.at[cidx.at[pl.ds(kk * IDXW, IDXW)]],
                                 ce.at[pl.ds(kk * IDXW, IDXW)], sem)
                for kk in range(NIDX)
            ]
            cp_t.wait()
            for cp in cps:
                cp.wait()

            def sg_body(sg, carry2):
                b0 = sg * 4
                accs = [jnp.zeros((16,), jnp.float32) for _ in range(5)]
                for boff in range(4):
                    b = b0 + boff
                    t0 = te[b, pl.ds(0, 16)]
                    t1 = te[b, pl.ds(16, 16)]
                    t2 = te[b, pl.ds(32, 16)]
                    t3 = te[b, pl.ds(48, 16)]
                    for j in range(CTX):
                        p = boff * CTX + j
                        row = b * CTX + j
                        s = (t0 * ce[row, pl.ds(64, 16)]
                             + t1 * ce[row, pl.ds(80, 16)]
                             + t2 * ce[row, pl.ds(96, 16)]
                             + t3 * ce[row, pl.ds(112, 16)])
                        v, l = divmod(p, 16)
                        accs[v] = jnp.where(lanes == l, jnp.sum(s), accs[v])
                for v in range(5):
                    ob[pl.ds(sg * 80 + v * 16, 16)] = accs[v]
                return carry2

            lax.fori_loop(0, CB // 4, sg_body, 0, unroll=False)

            out0 = wid * BPW * CTX + c * PAIRS
            pltpu.sync_copy(ob, out_hbm.at[pl.ds(out0, PAIRS)])
            return carry

        lax.fori_loop(0, NCHUNK, chunk_body, 0, unroll=False)

    return k(target_flat, ctx_flat, mtab)


def kernel(target, context, target_table, context_table):
    target_flat = target.reshape(B)
    ctx_flat = context.reshape(B * CTX)
    mtab = _tc_relayout(target_table.T, context_table.T)
    out_flat = _sc_skipgram(target_flat, ctx_flat, mtab)
    return out_flat.reshape(B, CTX)


# speedup vs baseline: 2.2671x; 1.0918x over previous
import functools

import jax
import jax.numpy as jnp
from jax import lax
from jax.experimental import pallas as pl
from jax.experimental.pallas import tpu as pltpu
from jax.experimental.pallas import tpu_sc as plsc

B = 16384
CTX = 20
D = 64
VOCAB = 1000000
NC = 2
NS = 16
NW = NC * NS
BPW = B // NW
CB = 16
NCHUNK = BPW // CB
PAIRS = CB * CTX
CSPLIT = ((0, 128), (128, 128), (256, 64))

TCOLS = 8192


def _tc_relayout(ttab_t, ctab_t):
    grid = (VOCAB + TCOLS - 1) // TCOLS

    def body(t_ref, c_ref, out_ref):
        eye = jnp.eye(D, dtype=jnp.bfloat16)
        tt = lax.dot_general(t_ref[...].astype(jnp.bfloat16), eye,
                             (((0,), (0,)), ((), ())),
                             preferred_element_type=jnp.float32)
        ct = lax.dot_general(c_ref[...].astype(jnp.bfloat16), eye,
                             (((0,), (0,)), ((), ())),
                             preferred_element_type=jnp.float32)
        out_ref[...] = jnp.concatenate([tt, ct], axis=1)

    return pl.pallas_call(
        body,
        grid=(grid,),
        in_specs=[pl.BlockSpec((D, TCOLS), lambda i: (0, i)),
                  pl.BlockSpec((D, TCOLS), lambda i: (0, i))],
        out_specs=pl.BlockSpec((TCOLS, 128), lambda i: (i, 0)),
        out_shape=jax.ShapeDtypeStruct((VOCAB, 128), jnp.float32),
    )(ttab_t, ctab_t)


def _sc_skipgram(target_flat, ctx_flat, mtab):
    mesh = plsc.VectorSubcoreMesh(core_axis_name="c", subcore_axis_name="s")

    @functools.partial(
        pl.kernel,
        mesh=mesh,
        compiler_params=pltpu.CompilerParams(
            needs_layout_passes=False, use_tc_tiling_on_sc=True),
        out_type=jax.ShapeDtypeStruct((B * CTX,), jnp.float32),
        scratch_types=[
            pltpu.VMEM((BPW,), jnp.int32),
            pltpu.VMEM((BPW * CTX,), jnp.int32),
            pltpu.VMEM((CB, 128), jnp.float32),
            pltpu.VMEM((CB, 128), jnp.float32),
            pltpu.VMEM((PAIRS, 128), jnp.float32),
            pltpu.VMEM((PAIRS, 128), jnp.float32),
            pltpu.VMEM((PAIRS,), jnp.float32),
            pltpu.SemaphoreType.DMA,
            pltpu.SemaphoreType.DMA,
        ],
    )
    def k(tgt_hbm, ctx_hbm, mtab_hbm, out_hbm,
          tidx, cidx, te0, te1, ce0, ce1, ob, sem0, sem1):
        wid = lax.axis_index("s") * NC + lax.axis_index("c")
        lanes = lax.iota(jnp.int32, 16)
        tes, ces, sems = (te0, te1), (ce0, ce1), (sem0, sem1)

        pltpu.sync_copy(tgt_hbm.at[pl.ds(wid * BPW, BPW)], tidx)
        pltpu.sync_copy(ctx_hbm.at[pl.ds(wid * BPW * CTX, BPW * CTX)], cidx)

        def issue(n, p):
            pltpu.async_copy(mtab_hbm.at[tidx.at[pl.ds(n * CB, CB)]],
                             tes[p], sems[p])
            for off, sz in CSPLIT:
                pltpu.async_copy(
                    mtab_hbm.at[cidx.at[pl.ds(n * PAIRS + off, sz)]],
                    ces[p].at[pl.ds(off, sz)], sems[p])

        def drain(p):
            pltpu.make_async_copy(mtab_hbm.at[pl.ds(0, CB)],
                                  tes[p], sems[p]).wait()
            pltpu.make_async_copy(mtab_hbm.at[pl.ds(0, PAIRS)],
                                  ces[p], sems[p]).wait()

        def compute(n, p):
            te, ce = tes[p], ces[p]

            def sg_body(sg, carry2):
                b0 = sg * 4
                accs = [jnp.zeros((16,), jnp.float32) for _ in range(5)]
                for boff in range(4):
                    b = b0 + boff
                    t0 = te[b, pl.ds(0, 16)]
                    t1 = te[b, pl.ds(16, 16)]
                    t2 = te[b, pl.ds(32, 16)]
                    t3 = te[b, pl.ds(48, 16)]
                    for j in range(CTX):
                        p2 = boff * CTX + j
                        row = b * CTX + j
                        s = (t0 * ce[row, pl.ds(64, 16)]
                             + t1 * ce[row, pl.ds(80, 16)]
                             + t2 * ce[row, pl.ds(96, 16)]
                             + t3 * ce[row, pl.ds(112, 16)])
                        v, l = divmod(p2, 16)
                        accs[v] = jnp.where(lanes == l, jnp.sum(s), accs[v])
                for v in range(5):
                    ob[pl.ds(sg * 80 + v * 16, 16)] = accs[v]
                return carry2

            lax.fori_loop(0, CB // 4, sg_body, 0, unroll=False)
            out0 = wid * BPW * CTX + n * PAIRS
            pltpu.sync_copy(ob, out_hbm.at[pl.ds(out0, PAIRS)])

        issue(0, 0)

        def pair_body(i, carry):
            n0 = i * 2
            issue(n0 + 1, 1)
            drain(0)
            compute(n0, 0)
            issue(jnp.minimum(n0 + 2, NCHUNK - 1), 0)
            drain(1)
            compute(n0 + 1, 1)
            return carry

        lax.fori_loop(0, NCHUNK // 2, pair_body, 0, unroll=False)
        drain(0)

    return k(target_flat, ctx_flat, mtab)


def kernel(target, context, target_table, context_table):
    target_flat = target.reshape(B)
    ctx_flat = context.reshape(B * CTX)
    mtab = _tc_relayout(target_table.T, context_table.T)
    out_flat = _sc_skipgram(target_flat, ctx_flat, mtab)
    return out_flat.reshape(B, CTX)


# speedup vs baseline: 2.5758x; 1.1362x over previous
import functools

import jax
import jax.numpy as jnp
from jax import lax
from jax.experimental import pallas as pl
from jax.experimental.pallas import tpu as pltpu
from jax.experimental.pallas import tpu_sc as plsc

B = 16384
CTX = 20
D = 64
VOCAB = 1000000
NC = 2
NS = 16
NW = NC * NS
BPW = B // NW
CB = 16
NCHUNK = BPW // CB
PAIRS = CB * CTX
CSPLIT = ((0, 128), (128, 128), (256, 64))

TCOLS = 16384


def _tc_relayout(ttab_t, ctab_t):
    grid = (VOCAB + TCOLS - 1) // TCOLS

    def body(t_ref, c_ref, out_ref):
        x = jnp.concatenate([t_ref[...], c_ref[...]],
                            axis=0).astype(jnp.bfloat16)
        eye = jnp.eye(2 * D, dtype=jnp.bfloat16)
        out_ref[...] = lax.dot_general(x, eye, (((0,), (0,)), ((), ())),
                                       preferred_element_type=jnp.float32)

    return pl.pallas_call(
        body,
        grid=(grid,),
        in_specs=[pl.BlockSpec((D, TCOLS), lambda i: (0, i)),
                  pl.BlockSpec((D, TCOLS), lambda i: (0, i))],
        out_specs=pl.BlockSpec((TCOLS, 128), lambda i: (i, 0)),
        out_shape=jax.ShapeDtypeStruct((VOCAB, 128), jnp.float32),
    )(ttab_t, ctab_t)


def _sc_skipgram(target_flat, ctx_flat, mtab):
    mesh = plsc.VectorSubcoreMesh(core_axis_name="c", subcore_axis_name="s")

    @functools.partial(
        pl.kernel,
        mesh=mesh,
        compiler_params=pltpu.CompilerParams(
            needs_layout_passes=False, use_tc_tiling_on_sc=True),
        out_type=jax.ShapeDtypeStruct((B * CTX,), jnp.float32),
        scratch_types=[
            pltpu.VMEM((BPW,), jnp.int32),
            pltpu.VMEM((BPW * CTX,), jnp.int32),
            pltpu.VMEM((CB, 128), jnp.float32),
            pltpu.VMEM((CB, 128), jnp.float32),
            pltpu.VMEM((PAIRS, 128), jnp.float32),
            pltpu.VMEM((PAIRS, 128), jnp.float32),
            pltpu.VMEM((PAIRS,), jnp.float32),
            pltpu.SemaphoreType.DMA,
            pltpu.SemaphoreType.DMA,
        ],
    )
    def k(tgt_hbm, ctx_hbm, mtab_hbm, out_hbm,
          tidx, cidx, te0, te1, ce0, ce1, ob, sem0, sem1):
        wid = lax.axis_index("s") * NC + lax.axis_index("c")
        lanes = lax.iota(jnp.int32, 16)
        tes, ces, sems = (te0, te1), (ce0, ce1), (sem0, sem1)

        pltpu.sync_copy(tgt_hbm.at[pl.ds(wid * BPW, BPW)], tidx)
        pltpu.sync_copy(ctx_hbm.at[pl.ds(wid * BPW * CTX, BPW * CTX)], cidx)

        def issue(n, p):
            pltpu.async_copy(mtab_hbm.at[tidx.at[pl.ds(n * CB, CB)]],
                             tes[p], sems[p])
            for off, sz in CSPLIT:
                pltpu.async_copy(
                    mtab_hbm.at[cidx.at[pl.ds(n * PAIRS + off, sz)]],
                    ces[p].at[pl.ds(off, sz)], sems[p])

        def drain(p):
            pltpu.make_async_copy(mtab_hbm.at[pl.ds(0, CB)],
                                  tes[p], sems[p]).wait()
            pltpu.make_async_copy(mtab_hbm.at[pl.ds(0, PAIRS)],
                                  ces[p], sems[p]).wait()

        def compute(n, p):
            te, ce = tes[p], ces[p]

            def sg_body(sg, carry2):
                b0 = sg * 4
                accs = [jnp.zeros((16,), jnp.float32) for _ in range(5)]
                for boff in range(4):
                    b = b0 + boff
                    t0 = te[b, pl.ds(0, 16)]
                    t1 = te[b, pl.ds(16, 16)]
                    t2 = te[b, pl.ds(32, 16)]
                    t3 = te[b, pl.ds(48, 16)]
                    for j in range(CTX):
                        p2 = boff * CTX + j
                        row = b * CTX + j
                        s = (t0 * ce[row, pl.ds(64, 16)]
                             + t1 * ce[row, pl.ds(80, 16)]
                             + t2 * ce[row, pl.ds(96, 16)]
                             + t3 * ce[row, pl.ds(112, 16)])
                        v, l = divmod(p2, 16)
                        accs[v] = jnp.where(lanes == l, jnp.sum(s), accs[v])
                for v in range(5):
                    ob[pl.ds(sg * 80 + v * 16, 16)] = accs[v]
                return carry2

            lax.fori_loop(0, CB // 4, sg_body, 0, unroll=False)
            out0 = wid * BPW * CTX + n * PAIRS
            pltpu.sync_copy(ob, out_hbm.at[pl.ds(out0, PAIRS)])

        issue(0, 0)

        def pair_body(i, carry):
            n0 = i * 2
            issue(n0 + 1, 1)
            drain(0)
            compute(n0, 0)
            issue(jnp.minimum(n0 + 2, NCHUNK - 1), 0)
            drain(1)
            compute(n0 + 1, 1)
            return carry

        lax.fori_loop(0, NCHUNK // 2, pair_body, 0, unroll=False)
        drain(0)

    return k(target_flat, ctx_flat, mtab)


def kernel(target, context, target_table, context_table):
    target_flat = target.reshape(B)
    ctx_flat = context.reshape(B * CTX)
    mtab = _tc_relayout(target_table.T, context_table.T)
    out_flat = _sc_skipgram(target_flat, ctx_flat, mtab)
    return out_flat.reshape(B, CTX)
